# baseline (speedup 1.0000x reference)
import jax
import jax.numpy as jnp
from jax.experimental import pallas as pl

B, N, T, H = 8, 325, 168, 64
N_BLK = 65


def _copy_body(node_ref, out_ref):
    out_ref[...] = node_ref[...]


def kernel(node_emb, hour_of_day, session_start, session_end, position_emb):
    return pl.pallas_call(
        _copy_body,
        grid=(B, N // N_BLK),
        in_specs=[
            pl.BlockSpec((1, N_BLK, T, H), lambda b, n: (b, n, 0, 0)),
        ],
        out_specs=pl.BlockSpec((1, N_BLK, T, H), lambda b, n: (b, n, 0, 0)),
        out_shape=jax.ShapeDtypeStruct((B, N, T, H), jnp.float32),
    )(node_emb)


# speedup vs baseline: 1.4881x; 1.4881x over previous
import functools

import jax
import jax.numpy as jnp
from jax import lax
from jax.experimental import pallas as pl
from jax.experimental.pallas import tpu as pltpu
from jax.experimental.pallas import tpu_sc as plsc

B, N, T, H = 8, 325, 168, 64
LN = 128
SL = T * H // LN
N_BLK = 25
CHUNKS = N // N_BLK
NC = B * CHUNKS
Q = 8
G = NC // Q


NW = 32
BT_PAD = 1536
R_PW = BT_PAD // NW
VL = 16


def _sc_gather_body(pos_hbm, ss_hbm, se_hbm, idx_hbm, out_hbm, aug_hbm,
                    table_v, ss_v, se_v, idx_v, adj_v, rows_v, sem):
    wid = lax.axis_index("s") * 2 + lax.axis_index("c")
    base = wid * R_PW
    pltpu.sync_copy(pos_hbm, table_v)
    pltpu.sync_copy(ss_hbm, ss_v)
    pltpu.sync_copy(se_hbm, se_v)
    pltpu.sync_copy(idx_hbm.at[pl.ds(base, R_PW)], idx_v)
    for k in range(H // VL):
        sl = pl.ds(k * VL, VL)
        table_v[0, sl] = table_v[0, sl] + ss_v[sl]
        table_v[23, sl] = table_v[23, sl] + se_v[sl]
    pltpu.sync_copy(table_v, aug_hbm.at[pl.ds(wid * 24, 24)])
    for g in range(R_PW // VL):
        sl = pl.ds(g * VL, VL)
        adj_v[sl] = idx_v[sl] + wid * 24
    pltpu.async_copy(aug_hbm.at[adj_v], rows_v, sem).wait()
    pltpu.sync_copy(rows_v, out_hbm.at[pl.ds(base, R_PW)])


def _stream_body(add_ref, node_ref, out_ref, ibuf, obuf, isem, osem):
    def in_copy(i, slot):
        b = i // CHUNKS
        c = jax.lax.rem(i, CHUNKS)
        return pltpu.make_async_copy(
            node_ref.at[b, pl.ds(c * N_BLK, N_BLK)], ibuf.at[slot],
            isem.at[slot])

    def out_copy(i, slot):
        b = i // CHUNKS
        c = jax.lax.rem(i, CHUNKS)
        return pltpu.make_async_copy(
            obuf.at[slot], out_ref.at[b, pl.ds(c * N_BLK, N_BLK)],
            osem.at[slot])

    for j in range(Q):
        in_copy(j, j).start()

    def group(g, carry):
        for j in range(Q):
            i = g * Q + j
            in_copy(i, j).wait()

            @pl.when(g > 0)
            def _():
                out_copy(i - Q, j).wait()

            b = i // CHUNKS
            obuf[j] = ibuf[j] + add_ref[b][None, :, :]
            out_copy(i, j).start()

            @pl.when(g < G - 1)
            def _():
                in_copy(i + Q, j).start()
        return carry

    jax.lax.fori_loop(0, G, group, 0)
    for j in range(Q):
        out_copy(NC - Q + j, j).wait()


def kernel(node_emb, hour_of_day, session_start, session_end, position_emb):
    idx = jnp.pad(hour_of_day.astype(jnp.int32).reshape(B * T),
                  (0, BT_PAD - B * T), constant_values=1)
    mesh = plsc.VectorSubcoreMesh(core_axis_name="c", subcore_axis_name="s")
    sc_gather = pl.kernel(
        _sc_gather_body,
        out_type=(jax.ShapeDtypeStruct((BT_PAD, 128), jnp.float32),
                  jax.ShapeDtypeStruct((NW * 24, 128), jnp.float32)),
        mesh=mesh,
        scratch_types=[
            pltpu.VMEM((24, 128), jnp.float32),
            pltpu.VMEM((H,), jnp.float32),
            pltpu.VMEM((H,), jnp.float32),
            pltpu.VMEM((R_PW,), jnp.int32),
            pltpu.VMEM((R_PW,), jnp.int32),
            pltpu.VMEM((R_PW, 128), jnp.float32),
            pltpu.SemaphoreType.DMA,
        ],
    )
    pos2 = jnp.tile(position_emb, (1, 2))
    add, _ = sc_gather(pos2, session_start, session_end, idx)
    add = add[:B * T, :H]

    node2 = node_emb.reshape(B, N, SL, LN)
    add2 = add.reshape(B, SL, LN)
    out2 = pl.pallas_call(
        _stream_body,
        in_specs=[
            pl.BlockSpec(memory_space=pltpu.VMEM),
            pl.BlockSpec(memory_space=pl.ANY),
        ],
        out_specs=pl.BlockSpec(memory_space=pl.ANY),
        out_shape=jax.ShapeDtypeStruct((B, N, SL, LN), jnp.float32),
        scratch_shapes=[
            pltpu.VMEM((Q, N_BLK, SL, LN), jnp.float32),
            pltpu.VMEM((Q, N_BLK, SL, LN), jnp.float32),
            pltpu.SemaphoreType.DMA((Q,)),
            pltpu.SemaphoreType.DMA((Q,)),
        ],
    )(add2, node2)
    return out2.reshape(B, N, T, H)


# speedup vs baseline: 1.5096x; 1.0144x over previous
import jax
import jax.numpy as jnp
from jax import lax
from jax.experimental import pallas as pl
from jax.experimental.pallas import tpu as pltpu
from jax.experimental.pallas import tpu_sc as plsc

B, N, T, H = 8, 325, 168, 64
C = 2
CH = T * H // C

NW = 32
BT_PAD = 1536
R_PW = BT_PAD // NW


def _sc_gather_body(tab_hbm, idx_hbm, out_hbm, idx_v, rows_v, sem):
    wid = lax.axis_index("s") * 2 + lax.axis_index("c")
    base = wid * R_PW
    pltpu.sync_copy(idx_hbm.at[pl.ds(base, R_PW)], idx_v)
    pltpu.async_copy(tab_hbm.at[idx_v], rows_v, sem).wait()
    pltpu.sync_copy(rows_v, out_hbm.at[pl.ds(base, R_PW)])


def _add_body(node_ref, add_ref, out_ref):
    out_ref[...] = node_ref[...] + add_ref[...]


def kernel(node_emb, hour_of_day, session_start, session_end, position_emb):
    table = (position_emb.at[0].add(session_start)
             .at[23].add(session_end))
    tab2 = jnp.tile(table, (1, 2))
    idx = jnp.pad(hour_of_day.astype(jnp.int32).reshape(B * T),
                  (0, BT_PAD - B * T), constant_values=1)
    mesh = plsc.VectorSubcoreMesh(core_axis_name="c", subcore_axis_name="s")
    sc_gather = pl.kernel(
        _sc_gather_body,
        out_type=jax.ShapeDtypeStruct((BT_PAD, 128), jnp.float32),
        mesh=mesh,
        scratch_types=[
            pltpu.VMEM((R_PW,), jnp.int32),
            pltpu.VMEM((R_PW, 128), jnp.float32),
            pltpu.SemaphoreType.DMA,
        ],
    )
    add = sc_gather(tab2, idx)[:B * T, :H]

    node2 = node_emb.reshape(B, N, T * H)
    add2 = add.reshape(B, 1, T * H)
    out2 = pl.pallas_call(
        _add_body,
        grid=(B, C),
        in_specs=[
            pl.BlockSpec((1, N, CH), lambda b, c: (b, 0, c)),
            pl.BlockSpec((1, 1, CH), lambda b, c: (b, 0, c)),
        ],
        out_specs=pl.BlockSpec((1, N, CH), lambda b, c: (b, 0, c)),
        out_shape=jax.ShapeDtypeStruct((B, N, T * H), jnp.float32),
    )(node2, add2)
    return out2.reshape(B, N, T, H)
